# Initial kernel scaffold; baseline (speedup 1.0000x reference)
import jax
import jax.numpy as jnp
from jax.experimental import pallas as pl


def kernel(x1, x2, edge_attr1, edge_attr2, W1, b1, W2, b2, edge_index1, edge_index2, batch):
    raise NotImplementedError("write your pallas kernel here")


# speedup vs baseline: 1.4622x; 1.4622x over previous
import functools
import jax
import jax.numpy as jnp
from jax import lax
from jax.experimental import pallas as pl
from jax.experimental.pallas import tpu as pltpu

N1 = 10000
E1 = 160000
E2 = 320000
D = 128
B = 128


def _edge_mlp_body(g_ref, e_ref, w1a_ref, w1b_ref, b1_ref, w2_ref, b2_ref, out_ref):
    h = jnp.dot(g_ref[...], w1a_ref[...], preferred_element_type=jnp.float32)
    h = h + jnp.dot(e_ref[...], w1b_ref[...], preferred_element_type=jnp.float32)
    h = jnp.maximum(h + b1_ref[...], 0.0)
    out_ref[...] = jnp.dot(h, w2_ref[...], preferred_element_type=jnp.float32) + b2_ref[...]


def _edge_mlp(gathered, edge_attr, W1a, W1b, b1, W2, b2, blk):
    n = gathered.shape[0]
    grid = n // blk
    return pl.pallas_call(
        _edge_mlp_body,
        grid=(grid,),
        in_specs=[
            pl.BlockSpec((blk, D), lambda i: (i, 0)),
            pl.BlockSpec((blk, D), lambda i: (i, 0)),
            pl.BlockSpec((D, 2 * D), lambda i: (0, 0)),
            pl.BlockSpec((D, 2 * D), lambda i: (0, 0)),
            pl.BlockSpec((1, 2 * D), lambda i: (0, 0)),
            pl.BlockSpec((2 * D, D), lambda i: (0, 0)),
            pl.BlockSpec((1, D), lambda i: (0, 0)),
        ],
        out_specs=pl.BlockSpec((blk, D), lambda i: (i, 0)),
        out_shape=jax.ShapeDtypeStruct((n, D), jnp.float32),
    )(gathered, edge_attr, W1a, W1b, b1, W2, b2)


def _proj_body(x_ref, w_ref, b_ref, out_ref):
    out_ref[...] = (
        jnp.dot(x_ref[...], w_ref[...], preferred_element_type=jnp.float32) + b_ref[...]
    )


def _node_proj(x, W1a, b1, blk):
    n = x.shape[0]
    return pl.pallas_call(
        _proj_body,
        grid=(n // blk,),
        in_specs=[
            pl.BlockSpec((blk, D), lambda i: (i, 0)),
            pl.BlockSpec((D, 2 * D), lambda i: (0, 0)),
            pl.BlockSpec((1, 2 * D), lambda i: (0, 0)),
        ],
        out_specs=pl.BlockSpec((blk, 2 * D), lambda i: (i, 0)),
        out_shape=jax.ShapeDtypeStruct((n, 2 * D), jnp.float32),
    )(x, W1a, b1)


def _pool_body(x1_ref, s1_ref, batch_ref, out_ref):
    h = x1_ref[...] + s1_ref[...]
    seg = batch_ref[0]
    onehot = (seg == lax.broadcasted_iota(jnp.int32, (B, seg.shape[1]), 0)).astype(
        jnp.float32
    )
    acc = jnp.dot(onehot, h, preferred_element_type=jnp.float32) * 0.5

    @pl.when(pl.program_id(0) == 0)
    def _init():
        out_ref[...] = acc

    @pl.when(pl.program_id(0) != 0)
    def _acc():
        out_ref[...] += acc


def _pool(x1, s1, batch, blk):
    n = x1.shape[0]
    batch3 = batch.reshape(n // blk, 1, blk)
    return pl.pallas_call(
        _pool_body,
        grid=(n // blk,),
        in_specs=[
            pl.BlockSpec((blk, D), lambda i: (i, 0)),
            pl.BlockSpec((blk, D), lambda i: (i, 0)),
            pl.BlockSpec((1, 1, blk), lambda i: (i, 0, 0)),
        ],
        out_specs=pl.BlockSpec((B, D), lambda i: (0, 0)),
        out_shape=jax.ShapeDtypeStruct((B, D), jnp.float32),
    )(x1, s1, batch3)


def kernel(x1, x2, edge_attr1, edge_attr2, W1, b1, W2, b2, edge_index1, edge_index2, batch):
    W1a = W1[:D]
    W1b = W1[D:]
    b1r = b1.reshape(1, 2 * D)
    b2r = b2.reshape(1, D)

    src2, dst2 = edge_index2[0], edge_index2[1]
    src1, dst1 = edge_index1[0], edge_index1[1]

    g2 = jnp.take(x2, src2, axis=0)
    m2 = _edge_mlp(g2, edge_attr2, W1a, W1b, b1r, W2, b2r, blk=2000)
    s2 = jax.ops.segment_sum(m2, dst2, num_segments=E1)
    e1 = edge_attr1 + s2

    g1 = jnp.take(x1, src1, axis=0)
    m1 = _edge_mlp(g1, e1, W1a, W1b, b1r, W2, b2r, blk=2000)
    s1 = jax.ops.segment_sum(m1, dst1, num_segments=N1)

    return _pool(x1, s1, batch, blk=2000)


# speedup vs baseline: 1.5659x; 1.0709x over previous
import functools
import jax
import jax.numpy as jnp
from jax import lax
from jax.experimental import pallas as pl
from jax.experimental.pallas import tpu as pltpu
from jax.experimental.pallas import tpu_sc as plsc

N1 = 10000
E1 = 160000
E2 = 320000
D = 128
B = 128

NC = 2
NS = 16
NW = NC * NS


def _sc_gather(table, idx, chunk):
    n = idx.shape[0]
    d = table.shape[1]
    b_per_w = n // NW
    n_chunks = b_per_w // chunk
    assert b_per_w % chunk == 0 and chunk % 8 == 0
    idx2 = idx.reshape(NW, b_per_w)
    mesh = plsc.VectorSubcoreMesh(core_axis_name="c", subcore_axis_name="s")

    @functools.partial(
        pl.kernel,
        out_type=jax.ShapeDtypeStruct((n, d), jnp.float32),
        mesh=mesh,
        scratch_types=[
            pltpu.VMEM((b_per_w,), jnp.int32),
            pltpu.VMEM((chunk, d), jnp.float32),
            pltpu.VMEM((chunk, d), jnp.float32),
            pltpu.SemaphoreType.DMA,
            pltpu.SemaphoreType.DMA,
        ],
    )
    def k(table_hbm, idx_hbm, out_hbm, idx_v, rows0, rows1, sem0, sem1):
        wid = lax.axis_index("s") * NC + lax.axis_index("c")
        base = wid * b_per_w
        pltpu.sync_copy(idx_hbm.at[wid], idx_v)
        pltpu.async_copy(table_hbm.at[idx_v.at[pl.ds(0, chunk)]], rows0, sem0)

        def step(j, rows_a, sem_a, rows_b, sem_b):
            @pl.when(j + 1 < n_chunks)
            def _():
                pltpu.async_copy(
                    table_hbm.at[idx_v.at[pl.ds((j + 1) * chunk, chunk)]], rows_b, sem_b
                )

            pltpu.make_async_copy(
                table_hbm.at[idx_v.at[pl.ds(j * chunk, chunk)]], rows_a, sem_a
            ).wait()
            pltpu.sync_copy(rows_a, out_hbm.at[pl.ds(base + j * chunk, chunk)])

        def body(t, _):
            step(2 * t, rows0, sem0, rows1, sem1)
            step(2 * t + 1, rows1, sem1, rows0, sem0)
            return 0

        lax.fori_loop(0, n_chunks // 2, body, 0, unroll=False)
        if n_chunks % 2 == 1:
            step(n_chunks - 1, rows0, sem0, rows1, sem1)

    return k(table, idx2)


def _sc_scatter_n1(m1s, dst1, x1s, chunk=200):
    H = D // 2
    e_per_t = E1 // NS
    n_chunks = e_per_t // chunk
    rows_per_tile = 624
    tail_base = NS * rows_per_tile
    tail_rows = N1 - tail_base
    assert e_per_t % chunk == 0
    mesh = plsc.VectorSubcoreMesh(core_axis_name="c", subcore_axis_name="s")

    @functools.partial(
        pl.kernel,
        out_type=jax.ShapeDtypeStruct((NC, N1, H), jnp.float32),
        mesh=mesh,
        scratch_types=[
            pltpu.VMEM((chunk,), jnp.int32),
            pltpu.VMEM((chunk,), jnp.int32),
            pltpu.VMEM((chunk, H), jnp.float32),
            pltpu.VMEM((chunk, H), jnp.float32),
            pltpu.VMEM_SHARED((N1, H), jnp.float32),
            pltpu.SemaphoreType.DMA,
            pltpu.SemaphoreType.DMA,
        ],
    )
    def k(m1_hbm, dst_hbm, x1_hbm, out_hbm, ix0, ix1, u0, u1, acc, sem0, sem1):
        c = lax.axis_index("c")
        s = lax.axis_index("s")
        ebase = pl.multiple_of(s * e_per_t, 8)
        stripe = pl.multiple_of(s * rows_per_tile, 8)
        last = s == NS - 1

        def hbm_to_spmem(src, dst, rows, base):
            full = rows // chunk
            for q in range(full):
                off = base + q * chunk
                pltpu.sync_copy(src.at[pl.ds(off, chunk)], u0)
                pltpu.sync_copy(u0, dst.at[pl.ds(off, chunk)])
            tail = rows - full * chunk
            if tail:
                off = base + full * chunk
                pltpu.sync_copy(src.at[pl.ds(off, tail)], u0.at[pl.ds(0, tail)])
                pltpu.sync_copy(u0.at[pl.ds(0, tail)], dst.at[pl.ds(off, tail)])

        def spmem_to_hbm(src, dst, rows, base):
            full = rows // chunk
            for q in range(full):
                off = base + q * chunk
                pltpu.sync_copy(src.at[pl.ds(off, chunk)], u0)
                pltpu.sync_copy(u0, dst.at[pl.ds(off, chunk)])
            tail = rows - full * chunk
            if tail:
                off = base + full * chunk
                pltpu.sync_copy(src.at[pl.ds(off, tail)], u0.at[pl.ds(0, tail)])
                pltpu.sync_copy(u0.at[pl.ds(0, tail)], dst.at[pl.ds(off, tail)])

        hbm_to_spmem(x1_hbm.at[c], acc, rows_per_tile, stripe)

        @pl.when(last)
        def _():
            hbm_to_spmem(x1_hbm.at[c], acc, tail_rows, tail_base)

        plsc.subcore_barrier()

        pltpu.async_copy(m1_hbm.at[c, pl.ds(ebase, chunk)], u0, sem0)
        pltpu.sync_copy(dst_hbm.at[pl.ds(ebase, chunk)], ix0)

        def step(j, ua, sem_a, ixa, ub, sem_b, ixb):
            @pl.when(j + 1 < n_chunks)
            def _():
                pltpu.async_copy(
                    m1_hbm.at[c, pl.ds(ebase + (j + 1) * chunk, chunk)], ub, sem_b
                )
                pltpu.sync_copy(dst_hbm.at[pl.ds(ebase + (j + 1) * chunk, chunk)], ixb)

            pltpu.make_async_copy(
                m1_hbm.at[c, pl.ds(ebase + j * chunk, chunk)], ua, sem_a
            ).wait()
            pltpu.sync_copy(ua, acc.at[ixa], add=True)

        def body(t, _):
            step(2 * t, u0, sem0, ix0, u1, sem1, ix1)
            step(2 * t + 1, u1, sem1, ix1, u0, sem0, ix0)
            return 0

        lax.fori_loop(0, n_chunks // 2, body, 0, unroll=False)
        if n_chunks % 2 == 1:
            step(n_chunks - 1, u0, sem0, ix0, u1, sem1, ix1)

        plsc.subcore_barrier()
        spmem_to_hbm(acc, out_hbm.at[c], rows_per_tile, stripe)

        @pl.when(last)
        def _():
            spmem_to_hbm(acc, out_hbm.at[c], tail_rows, tail_base)

    return k(m1s, dst1, x1s)


def _edge_mlp_body(g_ref, e_ref, w1a_ref, w1b_ref, b1_ref, w2_ref, b2_ref, out_ref):
    h = jnp.dot(g_ref[...], w1a_ref[...], preferred_element_type=jnp.float32)
    h = h + jnp.dot(e_ref[...], w1b_ref[...], preferred_element_type=jnp.float32)
    h = jnp.maximum(h + b1_ref[...], 0.0)
    res = jnp.dot(h, w2_ref[...], preferred_element_type=jnp.float32) + b2_ref[...]
    if out_ref.shape[0] == 2:
        out_ref[0] = res[:, : D // 2]
        out_ref[1] = res[:, D // 2 :]
    else:
        out_ref[...] = res


def _edge_mlp(gathered, edge_attr, W1a, W1b, b1, W2, b2, blk, split_out=False):
    n = gathered.shape[0]
    grid = n // blk
    if split_out:
        out_spec = pl.BlockSpec((2, blk, D // 2), lambda i: (0, i, 0))
        out_shape = jax.ShapeDtypeStruct((2, n, D // 2), jnp.float32)
    else:
        out_spec = pl.BlockSpec((blk, D), lambda i: (i, 0))
        out_shape = jax.ShapeDtypeStruct((n, D), jnp.float32)
    return pl.pallas_call(
        _edge_mlp_body,
        grid=(grid,),
        in_specs=[
            pl.BlockSpec((blk, D), lambda i: (i, 0)),
            pl.BlockSpec((blk, D), lambda i: (i, 0)),
            pl.BlockSpec((D, 2 * D), lambda i: (0, 0)),
            pl.BlockSpec((D, 2 * D), lambda i: (0, 0)),
            pl.BlockSpec((1, 2 * D), lambda i: (0, 0)),
            pl.BlockSpec((2 * D, D), lambda i: (0, 0)),
            pl.BlockSpec((1, D), lambda i: (0, 0)),
        ],
        out_specs=out_spec,
        out_shape=out_shape,
    )(gathered, edge_attr, W1a, W1b, b1, W2, b2)


def _proj_body(x_ref, w_ref, b_ref, out_ref):
    out_ref[...] = (
        jnp.dot(x_ref[...], w_ref[...], preferred_element_type=jnp.float32) + b_ref[...]
    )


def _node_proj(x, W1a, b1, blk):
    n = x.shape[0]
    return pl.pallas_call(
        _proj_body,
        grid=(n // blk,),
        in_specs=[
            pl.BlockSpec((blk, D), lambda i: (i, 0)),
            pl.BlockSpec((D, 2 * D), lambda i: (0, 0)),
            pl.BlockSpec((1, 2 * D), lambda i: (0, 0)),
        ],
        out_specs=pl.BlockSpec((blk, 2 * D), lambda i: (i, 0)),
        out_shape=jax.ShapeDtypeStruct((n, 2 * D), jnp.float32),
    )(x, W1a, b1)


def _pool_body(parts_ref, batch_ref, out_ref):
    p = parts_ref[...]
    h = jnp.concatenate([p[0], p[1]], axis=-1)
    seg = batch_ref[0]
    onehot = (seg == lax.broadcasted_iota(jnp.int32, (B, seg.shape[1]), 0)).astype(
        jnp.float32
    )
    acc = jnp.dot(onehot, h, preferred_element_type=jnp.float32) * 0.5

    @pl.when(pl.program_id(0) == 0)
    def _init():
        out_ref[...] = acc

    @pl.when(pl.program_id(0) != 0)
    def _acc():
        out_ref[...] += acc


def _pool(parts, batch, blk):
    n = parts.shape[1]
    batch3 = batch.reshape(n // blk, 1, blk)
    return pl.pallas_call(
        _pool_body,
        grid=(n // blk,),
        in_specs=[
            pl.BlockSpec((2, blk, D // 2), lambda i: (0, i, 0)),
            pl.BlockSpec((1, 1, blk), lambda i: (i, 0, 0)),
        ],
        out_specs=pl.BlockSpec((B, D), lambda i: (0, 0)),
        out_shape=jax.ShapeDtypeStruct((B, D), jnp.float32),
    )(parts, batch3)


def kernel(x1, x2, edge_attr1, edge_attr2, W1, b1, W2, b2, edge_index1, edge_index2, batch):
    W1a = W1[:D]
    W1b = W1[D:]
    b1r = b1.reshape(1, 2 * D)
    b2r = b2.reshape(1, D)

    src2, dst2 = edge_index2[0], edge_index2[1]
    src1, dst1 = edge_index1[0], edge_index1[1]

    g2 = _sc_gather(x2, src2, chunk=200)
    m2 = _edge_mlp(g2, edge_attr2, W1a, W1b, b1r, W2, b2r, blk=2000)
    s2 = jax.ops.segment_sum(m2, dst2, num_segments=E1)
    e1 = edge_attr1 + s2

    g1 = _sc_gather(x1, src1, chunk=200)
    BISECT = True
    if BISECT:
        m1 = _edge_mlp(g1, e1, W1a, W1b, b1r, W2, b2r, blk=2000)
        h = x1 + jax.ops.segment_sum(m1, dst1, num_segments=N1)
        parts = jnp.stack([h[:, : D // 2], h[:, D // 2 :]])
    else:
        m1s = _edge_mlp(g1, e1, W1a, W1b, b1r, W2, b2r, blk=2000, split_out=True)
        x1s = jnp.stack([x1[:, : D // 2], x1[:, D // 2 :]])
        parts = _sc_scatter_n1(m1s, dst1, x1s)

    return _pool(parts, batch, blk=2000)


# speedup vs baseline: 1.7477x; 1.1161x over previous
import functools
import jax
import jax.numpy as jnp
from jax import lax
from jax.experimental import pallas as pl
from jax.experimental.pallas import tpu as pltpu
from jax.experimental.pallas import tpu_sc as plsc

N1 = 10000
E1 = 160000
E2 = 320000
D = 128
B = 128

NC = 2
NS = 16
NW = NC * NS


def _sc_gather(table, idx, chunk):
    n = idx.shape[0]
    d = table.shape[1]
    b_per_w = n // NW
    n_chunks = b_per_w // chunk
    assert b_per_w % chunk == 0 and chunk % 8 == 0
    idx2 = idx.reshape(NW, b_per_w)
    mesh = plsc.VectorSubcoreMesh(core_axis_name="c", subcore_axis_name="s")

    @functools.partial(
        pl.kernel,
        out_type=jax.ShapeDtypeStruct((n, d), jnp.float32),
        mesh=mesh,
        scratch_types=[
            pltpu.VMEM((b_per_w,), jnp.int32),
            pltpu.VMEM((chunk, d), jnp.float32),
            pltpu.VMEM((chunk, d), jnp.float32),
            pltpu.SemaphoreType.DMA,
            pltpu.SemaphoreType.DMA,
        ],
    )
    def k(table_hbm, idx_hbm, out_hbm, idx_v, rows0, rows1, sem0, sem1):
        wid = lax.axis_index("s") * NC + lax.axis_index("c")
        base = wid * b_per_w
        pltpu.sync_copy(idx_hbm.at[wid], idx_v)
        pltpu.async_copy(table_hbm.at[idx_v.at[pl.ds(0, chunk)]], rows0, sem0)

        def step(j, rows_a, sem_a, rows_b, sem_b):
            @pl.when(j + 1 < n_chunks)
            def _():
                pltpu.async_copy(
                    table_hbm.at[idx_v.at[pl.ds((j + 1) * chunk, chunk)]], rows_b, sem_b
                )

            pltpu.make_async_copy(
                table_hbm.at[idx_v.at[pl.ds(j * chunk, chunk)]], rows_a, sem_a
            ).wait()
            pltpu.sync_copy(rows_a, out_hbm.at[pl.ds(base + j * chunk, chunk)])

        def body(t, _):
            step(2 * t, rows0, sem0, rows1, sem1)
            step(2 * t + 1, rows1, sem1, rows0, sem0)
            return 0

        lax.fori_loop(0, n_chunks // 2, body, 0, unroll=False)
        if n_chunks % 2 == 1:
            step(n_chunks - 1, rows0, sem0, rows1, sem1)

    return k(table, idx2)


def _sc_scatter_n1(m1s, dst1, x1s, chunk=200, do_add=True, do_scatter=True):
    H = D // 2
    e_per_t = E1 // NS
    n_chunks = e_per_t // chunk
    rows_per_tile = 624
    tail_base = NS * rows_per_tile
    tail_rows = N1 - tail_base
    assert e_per_t % chunk == 0
    mesh = plsc.VectorSubcoreMesh(core_axis_name="c", subcore_axis_name="s")

    @functools.partial(
        pl.kernel,
        out_type=jax.ShapeDtypeStruct((NC, N1, H), jnp.float32),
        mesh=mesh,
        scratch_types=[
            pltpu.VMEM((chunk,), jnp.int32),
            pltpu.VMEM((chunk,), jnp.int32),
            pltpu.VMEM((chunk, H), jnp.float32),
            pltpu.VMEM((chunk, H), jnp.float32),
            pltpu.VMEM_SHARED((N1, H), jnp.float32),
            pltpu.SemaphoreType.DMA,
            pltpu.SemaphoreType.DMA,
        ],
    )
    def k(m1_hbm, dst_hbm, x1_hbm, out_hbm, ix0, ix1, u0, u1, acc, sem0, sem1):
        c = lax.axis_index("c")
        s = lax.axis_index("s")
        ebase = pl.multiple_of(s * e_per_t, 8)
        stripe = pl.multiple_of(s * rows_per_tile, 8)
        last = s == NS - 1

        def hbm_to_spmem(src, dst, rows, base):
            full = rows // chunk
            for q in range(full):
                off = base + q * chunk
                pltpu.sync_copy(src.at[pl.ds(off, chunk)], u0)
                pltpu.sync_copy(u0, dst.at[pl.ds(off, chunk)])
            tail = rows - full * chunk
            if tail:
                off = base + full * chunk
                pltpu.sync_copy(src.at[pl.ds(off, tail)], u0.at[pl.ds(0, tail)])
                pltpu.sync_copy(u0.at[pl.ds(0, tail)], dst.at[pl.ds(off, tail)])

        def spmem_to_hbm(src, dst, rows, base):
            full = rows // chunk
            for q in range(full):
                off = base + q * chunk
                pltpu.sync_copy(src.at[pl.ds(off, chunk)], u0)
                pltpu.sync_copy(u0, dst.at[pl.ds(off, chunk)])
            tail = rows - full * chunk
            if tail:
                off = base + full * chunk
                pltpu.sync_copy(src.at[pl.ds(off, tail)], u0.at[pl.ds(0, tail)])
                pltpu.sync_copy(u0.at[pl.ds(0, tail)], dst.at[pl.ds(off, tail)])

        hbm_to_spmem(x1_hbm.at[c], acc, rows_per_tile, stripe)

        @pl.when(last)
        def _():
            hbm_to_spmem(x1_hbm.at[c], acc, tail_rows, tail_base)

        plsc.subcore_barrier()

        if do_scatter == "async_probe":
            pltpu.async_copy(m1_hbm.at[c, pl.ds(ebase, chunk)], u0, sem0)

            def astep(j, ua, sem_a, ub, sem_b):
                @pl.when(j + 1 < n_chunks)
                def _():
                    pltpu.async_copy(
                        m1_hbm.at[c, pl.ds(ebase + (j + 1) * chunk, chunk)], ub, sem_b
                    )

                pltpu.make_async_copy(
                    m1_hbm.at[c, pl.ds(ebase + j * chunk, chunk)], ua, sem_a
                ).wait()
                pltpu.sync_copy(ua, acc.at[pl.ds(stripe, chunk)])

            def abody(t, _):
                astep(2 * t, u0, sem0, u1, sem1)
                astep(2 * t + 1, u1, sem1, u0, sem0)
                return 0

            lax.fori_loop(0, n_chunks // 2, abody, 0, unroll=False)
            if n_chunks % 2 == 1:
                astep(n_chunks - 1, u0, sem0, u1, sem1)
        elif do_scatter == "sync_real":
            def sbody(j, _):
                pltpu.sync_copy(dst_hbm.at[pl.ds(ebase + j * chunk, chunk)], ix0)
                pltpu.sync_copy(m1_hbm.at[c, pl.ds(ebase + j * chunk, chunk)], u0)
                pltpu.sync_copy(u0, acc.at[ix0], add=True)
                return 0

            lax.fori_loop(0, n_chunks, sbody, 0, unroll=False)
        elif do_scatter:
            pltpu.async_copy(m1_hbm.at[c, pl.ds(ebase, chunk)], u0, sem0)
            pltpu.sync_copy(dst_hbm.at[pl.ds(ebase, chunk)], ix0)

            def step(j, ua, sem_a, ixa, ub, sem_b, ixb):
                @pl.when(j + 1 < n_chunks)
                def _():
                    pltpu.async_copy(
                        m1_hbm.at[c, pl.ds(ebase + (j + 1) * chunk, chunk)], ub, sem_b
                    )
                    pltpu.sync_copy(
                        dst_hbm.at[pl.ds(ebase + (j + 1) * chunk, chunk)], ixb
                    )

                pltpu.make_async_copy(
                    m1_hbm.at[c, pl.ds(ebase + j * chunk, chunk)], ua, sem_a
                ).wait()
                if do_add is None:
                    pltpu.sync_copy(ua, acc.at[pl.ds(stripe, chunk)])
                else:
                    pltpu.sync_copy(ua, acc.at[ixa], add=do_add)

            def body(t, _):
                step(2 * t, u0, sem0, ix0, u1, sem1, ix1)
                step(2 * t + 1, u1, sem1, ix1, u0, sem0, ix0)
                return 0

            lax.fori_loop(0, n_chunks // 2, body, 0, unroll=False)
            if n_chunks % 2 == 1:
                step(n_chunks - 1, u0, sem0, ix0, u1, sem1, ix1)

        plsc.subcore_barrier()
        spmem_to_hbm(acc, out_hbm.at[c], rows_per_tile, stripe)

        @pl.when(last)
        def _():
            spmem_to_hbm(acc, out_hbm.at[c], tail_rows, tail_base)

    return k(m1s, dst1, x1s)


def _edge_mlp_body(g_ref, e_ref, w1a_ref, w1b_ref, b1_ref, w2_ref, b2_ref, out_ref):
    h = jnp.dot(g_ref[...], w1a_ref[...], preferred_element_type=jnp.float32)
    h = h + jnp.dot(e_ref[...], w1b_ref[...], preferred_element_type=jnp.float32)
    h = jnp.maximum(h + b1_ref[...], 0.0)
    res = jnp.dot(h, w2_ref[...], preferred_element_type=jnp.float32) + b2_ref[...]
    if out_ref.shape[0] == 2:
        out_ref[0] = res[:, : D // 2]
        out_ref[1] = res[:, D // 2 :]
    else:
        out_ref[...] = res


def _edge_mlp(gathered, edge_attr, W1a, W1b, b1, W2, b2, blk, split_out=False):
    n = gathered.shape[0]
    grid = n // blk
    if split_out:
        out_spec = pl.BlockSpec((2, blk, D // 2), lambda i: (0, i, 0))
        out_shape = jax.ShapeDtypeStruct((2, n, D // 2), jnp.float32)
    else:
        out_spec = pl.BlockSpec((blk, D), lambda i: (i, 0))
        out_shape = jax.ShapeDtypeStruct((n, D), jnp.float32)
    return pl.pallas_call(
        _edge_mlp_body,
        grid=(grid,),
        in_specs=[
            pl.BlockSpec((blk, D), lambda i: (i, 0)),
            pl.BlockSpec((blk, D), lambda i: (i, 0)),
            pl.BlockSpec((D, 2 * D), lambda i: (0, 0)),
            pl.BlockSpec((D, 2 * D), lambda i: (0, 0)),
            pl.BlockSpec((1, 2 * D), lambda i: (0, 0)),
            pl.BlockSpec((2 * D, D), lambda i: (0, 0)),
            pl.BlockSpec((1, D), lambda i: (0, 0)),
        ],
        out_specs=out_spec,
        out_shape=out_shape,
    )(gathered, edge_attr, W1a, W1b, b1, W2, b2)


def _proj_body(x_ref, w_ref, b_ref, out_ref):
    out_ref[...] = (
        jnp.dot(x_ref[...], w_ref[...], preferred_element_type=jnp.float32) + b_ref[...]
    )


def _node_proj(x, W1a, b1, blk):
    n = x.shape[0]
    return pl.pallas_call(
        _proj_body,
        grid=(n // blk,),
        in_specs=[
            pl.BlockSpec((blk, D), lambda i: (i, 0)),
            pl.BlockSpec((D, 2 * D), lambda i: (0, 0)),
            pl.BlockSpec((1, 2 * D), lambda i: (0, 0)),
        ],
        out_specs=pl.BlockSpec((blk, 2 * D), lambda i: (i, 0)),
        out_shape=jax.ShapeDtypeStruct((n, 2 * D), jnp.float32),
    )(x, W1a, b1)


def _pool_body(parts_ref, batch_ref, out_ref):
    p = parts_ref[...]
    h = jnp.concatenate([p[0], p[1]], axis=-1)
    seg = batch_ref[0]
    onehot = (seg == lax.broadcasted_iota(jnp.int32, (B, seg.shape[1]), 0)).astype(
        jnp.float32
    )
    acc = jnp.dot(onehot, h, preferred_element_type=jnp.float32) * 0.5

    @pl.when(pl.program_id(0) == 0)
    def _init():
        out_ref[...] = acc

    @pl.when(pl.program_id(0) != 0)
    def _acc():
        out_ref[...] += acc


def _pool(parts, batch, blk):
    n = parts.shape[1]
    batch3 = batch.reshape(n // blk, 1, blk)
    return pl.pallas_call(
        _pool_body,
        grid=(n // blk,),
        in_specs=[
            pl.BlockSpec((2, blk, D // 2), lambda i: (0, i, 0)),
            pl.BlockSpec((1, 1, blk), lambda i: (i, 0, 0)),
        ],
        out_specs=pl.BlockSpec((B, D), lambda i: (0, 0)),
        out_shape=jax.ShapeDtypeStruct((B, D), jnp.float32),
    )(parts, batch3)


def kernel(x1, x2, edge_attr1, edge_attr2, W1, b1, W2, b2, edge_index1, edge_index2, batch):
    W1a = W1[:D]
    W1b = W1[D:]
    b1r = b1.reshape(1, 2 * D)
    b2r = b2.reshape(1, D)

    src2, dst2 = edge_index2[0], edge_index2[1]
    src1, dst1 = edge_index1[0], edge_index1[1]

    g2 = _sc_gather(x2, src2, chunk=200)
    m2 = _edge_mlp(g2, edge_attr2, W1a, W1b, b1r, W2, b2r, blk=2000)
    s2 = jax.ops.segment_sum(m2, dst2, num_segments=E1)
    e1 = edge_attr1 + s2

    g1 = _sc_gather(x1, src1, chunk=200)
    m1s = _edge_mlp(g1, e1, W1a, W1b, b1r, W2, b2r, blk=2000, split_out=True)
    x1s = jnp.stack([x1[:, : D // 2], x1[:, D // 2 :]])
    parts = _sc_scatter_n1(m1s, dst1, x1s, do_scatter="sync_real")

    return _pool(parts, batch, blk=2000)
